# Initial kernel scaffold; baseline (speedup 1.0000x reference)
import jax
import jax.numpy as jnp
from jax.experimental import pallas as pl


def kernel(x, edge_index, params):
    raise NotImplementedError("write your pallas kernel here")


# speedup vs baseline: 1.2380x; 1.2380x over previous
import functools

import jax
import jax.numpy as jnp
from jax.experimental import pallas as pl
from jax.experimental.pallas import tpu as pltpu

F32 = jnp.float32
_ROWS = 2000


def _dot(a, b):
    return jnp.dot(a, b, preferred_element_type=F32)


def _pre_body(x_ref, wt_ref, wb_ref, pb_ref, b_out, c_out):
    x = x_ref[...]
    b_out[...] = _dot(x, wb_ref[...])
    c_out[...] = _dot(x, wt_ref[...]) + pb_ref[...]


def _pre(x, pW, pb):
    n, fi = x.shape
    fo = pW.shape[1]
    wt, wb = pW[:fi], pW[fi:]
    grid = (n // _ROWS,)
    return pl.pallas_call(
        _pre_body,
        grid=grid,
        in_specs=[
            pl.BlockSpec((_ROWS, fi), lambda i: (i, 0)),
            pl.BlockSpec((fi, fo), lambda i: (0, 0)),
            pl.BlockSpec((fi, fo), lambda i: (0, 0)),
            pl.BlockSpec((1, fo), lambda i: (0, 0)),
        ],
        out_specs=[
            pl.BlockSpec((_ROWS, fo), lambda i: (i, 0)),
            pl.BlockSpec((_ROWS, fo), lambda i: (i, 0)),
        ],
        out_shape=[
            jax.ShapeDtypeStruct((n, fo), F32),
            jax.ShapeDtypeStruct((n, fo), F32),
        ],
    )(x, wt, wb, pb.reshape(1, fo))


def _combine_body(x_ref, s_ref, s2_ref, mn_ref, mx_ref, c_ref, deg_ref,
                  sc_ref, qx_ref, q0_ref, q1_ref, q2_ref, q3_ref, q4_ref,
                  qb_ref, lw_ref, lb_ref, out_ref):
    deg = deg_ref[...]
    safe = jnp.maximum(deg, 1.0)
    c = c_ref[...]
    s = s_ref[...]
    ssum = deg * c + s
    mean = ssum / safe
    mean2 = (deg * c * c + 2.0 * c * s + s2_ref[...]) / safe
    std = jnp.sqrt(jax.nn.relu(mean2 - mean * mean) + 1e-5)
    has = deg > 0
    mn = jnp.where(has, c + mn_ref[...], 0.0)
    mx = jnp.where(has, c + mx_ref[...], 0.0)
    agg = jnp.concatenate([ssum, mean, mn, mx, std], axis=-1)
    avg_log = sc_ref[0, 0]
    avg_lin = sc_ref[0, 1]
    d = safe
    ld = jnp.log(d + 1.0)
    y = (_dot(x_ref[...], qx_ref[...]) + _dot(agg, q0_ref[...])
         + (ld / avg_log) * _dot(agg, q1_ref[...])
         + (avg_log / ld) * _dot(agg, q2_ref[...])
         + (d / avg_lin) * _dot(agg, q3_ref[...])
         + (avg_lin / d) * _dot(agg, q4_ref[...])
         + qb_ref[...])
    out_ref[...] = _dot(y, lw_ref[...]) + lb_ref[...]


def _combine(x, S, S2, Mn, Mx, C, deg2d, scalars, qW, qb, lW, lb):
    n, fi = x.shape
    fo = S.shape[1]
    po = qW.shape[1]
    fl = lW.shape[1]
    qx = qW[:fi]
    qs = [qW[fi + k * 5 * fo: fi + (k + 1) * 5 * fo] for k in range(5)]
    grid = (n // _ROWS,)
    full = lambda a, b: pl.BlockSpec((a, b), lambda i: (0, 0))
    rows = lambda w: pl.BlockSpec((_ROWS, w), lambda i: (i, 0))
    return pl.pallas_call(
        _combine_body,
        grid=grid,
        in_specs=[
            rows(fi), rows(fo), rows(fo), rows(fo), rows(fo), rows(fo),
            rows(1), full(1, 2), full(fi, po),
            full(5 * fo, po), full(5 * fo, po), full(5 * fo, po),
            full(5 * fo, po), full(5 * fo, po),
            full(1, po), full(po, fl), full(1, fl),
        ],
        out_specs=rows(fl),
        out_shape=jax.ShapeDtypeStruct((n, fl), F32),
    )(x, S, S2, Mn, Mx, C, deg2d, scalars, qx, *qs,
      qb.reshape(1, po), lW, lb.reshape(1, fl))


def _head_body(x_ref, x1_ref, x2_ref, x3_ref, nw_ref, nb_ref, a1w_ref,
               a1b_ref, a2w_ref, a2b_ref, f1_ref, f1b_ref, f2_ref, f2b_ref,
               f3_ref, f3b_ref, out_ref):
    gg1 = jax.nn.relu(_dot(x_ref[...], nw_ref[...]) + nb_ref[...])
    xa1 = jax.nn.relu(_dot(x1_ref[...], a1w_ref[...]) + a1b_ref[...])
    xa2 = jax.nn.relu(_dot(x2_ref[...], a2w_ref[...]) + a2b_ref[...])
    xf = jnp.concatenate([gg1, x3_ref[...], xa1, xa2], axis=1)
    xf = jax.nn.relu(_dot(xf, f1_ref[...]) + f1b_ref[...])
    xf = jax.nn.relu(_dot(xf, f2_ref[...]) + f2b_ref[...])
    xf = _dot(xf, f3_ref[...]) + f3b_ref[...]
    out_ref[...] = 1.0 / (1.0 + jnp.exp(-xf))


def _head(x, x1, x2, x3, p):
    n = x.shape[0]
    grid = (n // _ROWS,)
    full = lambda a, b: pl.BlockSpec((a, b), lambda i: (0, 0))
    rows = lambda w: pl.BlockSpec((_ROWS, w), lambda i: (i, 0))
    args = [
        (x, rows(128)), (x1, rows(32)), (x2, rows(64)), (x3, rows(128)),
        (p['node1_W'], full(128, 32)), (p['node1_b'].reshape(1, 32), full(1, 32)),
        (p['after1_W'], full(32, 64)), (p['after1_b'].reshape(1, 64), full(1, 64)),
        (p['after3_W'], full(64, 128)), (p['after3_b'].reshape(1, 128), full(1, 128)),
        (p['fin1_W'], full(352, 300)), (p['fin1_b'].reshape(1, 300), full(1, 300)),
        (p['fin2_W'], full(300, 124)), (p['fin2_b'].reshape(1, 124), full(1, 124)),
        (p['fin3_W'], full(124, 1)), (p['fin3_b'].reshape(1, 1), full(1, 1)),
    ]
    return pl.pallas_call(
        _head_body,
        grid=grid,
        in_specs=[s for _, s in args],
        out_specs=rows(1),
        out_shape=jax.ShapeDtypeStruct((n, 1), F32),
    )(*[a for a, _ in args])


def _segment_reduce(B, src, dst, n):
    Bs = B[src]
    S = jax.ops.segment_sum(Bs, dst, num_segments=n)
    S2 = jax.ops.segment_sum(Bs * Bs, dst, num_segments=n)
    Mn = jax.ops.segment_min(Bs, dst, num_segments=n)
    Mx = jax.ops.segment_max(Bs, dst, num_segments=n)
    return S, S2, Mn, Mx


def _layer(x, src, dst, deg2d, scalars, p, tag):
    B, C = _pre(x, p[tag[0] + '_W'], p[tag[0] + '_b'])
    S, S2, Mn, Mx = _segment_reduce(B, src, dst, x.shape[0])
    return _combine(x, S, S2, Mn, Mx, C, deg2d, scalars,
                    p[tag[1] + '_W'], p[tag[1] + '_b'],
                    p[tag[2] + '_W'], p[tag[2] + '_b'])


def kernel(x, edge_index, params):
    p = params
    src = edge_index[0]
    dst = edge_index[1]
    n = x.shape[0]
    deg = jax.ops.segment_sum(jnp.ones(src.shape[0], dtype=F32), dst,
                              num_segments=n)
    avg_lin = jnp.mean(deg)
    avg_log = jnp.mean(jnp.log(deg + 1.0))
    scalars = jnp.stack([avg_log, avg_lin]).reshape(1, 2)
    deg2d = deg.reshape(n, 1)
    x1 = _layer(x, src, dst, deg2d, scalars, p, ('pre1', 'post1', 'lin1'))
    x2 = _layer(x1, src, dst, deg2d, scalars, p, ('pre2', 'post2', 'lin2'))
    x3 = _layer(x2, src, dst, deg2d, scalars, p, ('pre3', 'post3', 'lin3'))
    return _head(x, x1, x2, x3, p)
